# baseline (speedup 1.0000x reference)
import functools

import jax
import jax.numpy as jnp
from jax import lax
from jax.experimental import pallas as pl
from jax.experimental.pallas import tpu as pltpu
from jax.experimental.pallas import tpu_sc as plsc

_NC = 2
_NS = 16
_NW = _NC * _NS
_B = 16384
_D = 64
_BPW = _B // _NW
_L = 16
_CHUNK = 128
_NCHUNK = _BPW // _CHUNK


def _build():
    mesh = plsc.VectorSubcoreMesh(core_axis_name="c", subcore_axis_name="s")

    @functools.partial(
        pl.kernel,
        out_type=jax.ShapeDtypeStruct((_B,), jnp.float32),
        mesh=mesh,
        scratch_types=[
            pltpu.VMEM((_NCHUNK, _CHUNK), jnp.int32),
            pltpu.VMEM((_NCHUNK, _CHUNK), jnp.int32),
            pltpu.VMEM((_BPW, _D), jnp.bfloat16),
            pltpu.VMEM((_BPW, _D), jnp.bfloat16),
            pltpu.VMEM((_L, 17), jnp.float32),
            pltpu.VMEM((_BPW,), jnp.float32),
            pltpu.SemaphoreType.DMA,
            pltpu.SemaphoreType.DMA,
            pltpu.SemaphoreType.DMA,
            pltpu.SemaphoreType.DMA,
        ],
        compiler_params=pltpu.CompilerParams(
            needs_layout_passes=False, use_tc_tiling_on_sc=False
        ),
    )
    def run(user_h, item_h, ut_h, it_h, out_h, uidx, iidx, urows, irows, smat,
            outv, *sems):
        wid = lax.axis_index("s") * _NC + lax.axis_index("c")
        base = wid * _BPW

        for j in range(_NCHUNK):
            pltpu.sync_copy(user_h.at[pl.ds(base + j * _CHUNK, _CHUNK)], uidx.at[j])
            pltpu.sync_copy(item_h.at[pl.ds(base + j * _CHUNK, _CHUNK)], iidx.at[j])

        handles = []
        for j in range(_NCHUNK):
            cu = pltpu.async_copy(
                ut_h.at[uidx.at[j]], urows.at[pl.ds(j * _CHUNK, _CHUNK)], sems[j]
            )
            ci = pltpu.async_copy(
                it_h.at[iidx.at[j]], irows.at[pl.ds(j * _CHUNK, _CHUNK)], sems[j]
            )
            handles.append((cu, ci))

        lanes = lax.iota(jnp.int32, _L)

        def group(g):
            rbase = g * _L
            for r in range(_L):
                s = None
                for c in range(_D // 32):
                    u2 = urows[rbase + r, pl.ds(c * 32, 32)]
                    v2 = irows[rbase + r, pl.ds(c * 32, 32)]
                    ua, ub = plsc.unpack(u2, format=plsc.PackFormat.INTERLEAVED)
                    va, vb = plsc.unpack(v2, format=plsc.PackFormat.INTERLEAVED)
                    p = ua * va + ub * vb
                    s = p if s is None else s + p
                smat[r, pl.ds(0, _L)] = s
            acc = jnp.zeros((_L,), jnp.float32)
            for k in range(_L):
                col = plsc.load_gather(smat, [lanes, jnp.full((_L,), k, jnp.int32)])
                acc = acc + col
            outv[pl.ds(rbase, _L)] = acc

        for j in range(_NCHUNK):
            cu, ci = handles[j]
            cu.wait()
            ci.wait()

            def chunk_body(gg, carry, j=j):
                group(j * (_CHUNK // _L) + gg)
                return carry

            lax.fori_loop(0, _CHUNK // _L, chunk_body, 0)

        pltpu.sync_copy(outv, out_h.at[pl.ds(base, _BPW)])

    return run


_KERNEL = _build()


def kernel(user, item, user_table, item_table):
    return _KERNEL(
        user.astype(jnp.int32),
        item.astype(jnp.int32),
        user_table.astype(jnp.bfloat16),
        item_table.astype(jnp.bfloat16),
    )


# speedup vs baseline: 1.3098x; 1.3098x over previous
import functools

import jax
import jax.numpy as jnp
from jax import lax
from jax.experimental import pallas as pl
from jax.experimental.pallas import tpu as pltpu
from jax.experimental.pallas import tpu_sc as plsc

_NC = 2
_NS = 16
_NW = _NC * _NS
_B = 16384
_D = 64
_BPW = _B // _NW
_L = 16
_CHUNK = 128
_NCHUNK = _BPW // _CHUNK


def _build():
    mesh = plsc.VectorSubcoreMesh(core_axis_name="c", subcore_axis_name="s")

    @functools.partial(
        pl.kernel,
        out_type=jax.ShapeDtypeStruct((_B,), jnp.float32),
        mesh=mesh,
        scratch_types=[
            pltpu.VMEM((_NCHUNK, _CHUNK), jnp.int32),
            pltpu.VMEM((_NCHUNK, _CHUNK), jnp.int32),
            pltpu.VMEM((_BPW, _D), jnp.float32),
            pltpu.VMEM((_BPW, _D), jnp.float32),
            pltpu.VMEM((_L, 17), jnp.float32),
            pltpu.VMEM((_BPW,), jnp.float32),
            pltpu.SemaphoreType.DMA,
            pltpu.SemaphoreType.DMA,
            pltpu.SemaphoreType.DMA,
            pltpu.SemaphoreType.DMA,
        ],
        compiler_params=pltpu.CompilerParams(
            needs_layout_passes=False, use_tc_tiling_on_sc=False
        ),
    )
    def run(user_h, item_h, ut_h, it_h, out_h, uidx, iidx, urows, irows, smat,
            outv, *sems):
        wid = lax.axis_index("s") * _NC + lax.axis_index("c")
        base = wid * _BPW

        for j in range(_NCHUNK):
            pltpu.sync_copy(user_h.at[pl.ds(base + j * _CHUNK, _CHUNK)], uidx.at[j])
            pltpu.sync_copy(item_h.at[pl.ds(base + j * _CHUNK, _CHUNK)], iidx.at[j])

        handles = []
        for j in range(_NCHUNK):
            cu = pltpu.async_copy(
                ut_h.at[uidx.at[j]], urows.at[pl.ds(j * _CHUNK, _CHUNK)], sems[j]
            )
            ci = pltpu.async_copy(
                it_h.at[iidx.at[j]], irows.at[pl.ds(j * _CHUNK, _CHUNK)], sems[j]
            )
            handles.append((cu, ci))

        lanes = lax.iota(jnp.int32, _L)

        def group(g):
            rbase = g * _L
            for r in range(_L):
                s = None
                for c in range(_D // _L):
                    u = urows[rbase + r, pl.ds(c * _L, _L)]
                    v = irows[rbase + r, pl.ds(c * _L, _L)]
                    s = u * v if s is None else s + u * v
                smat[r, pl.ds(0, _L)] = s
            acc = jnp.zeros((_L,), jnp.float32)
            for k in range(_L):
                col = plsc.load_gather(smat, [lanes, jnp.full((_L,), k, jnp.int32)])
                acc = acc + col
            outv[pl.ds(rbase, _L)] = acc

        for j in range(_NCHUNK):
            cu, ci = handles[j]
            cu.wait()
            ci.wait()

            def chunk_body(gg, carry, j=j):
                group(j * (_CHUNK // _L) + gg)
                return carry

            lax.fori_loop(0, _CHUNK // _L, chunk_body, 0)

        pltpu.sync_copy(outv, out_h.at[pl.ds(base, _BPW)])

    return run


_KERNEL = _build()


def kernel(user, item, user_table, item_table):
    return _KERNEL(
        user.astype(jnp.int32),
        item.astype(jnp.int32),
        user_table,
        item_table,
    )


# speedup vs baseline: 1.3184x; 1.0065x over previous
import functools

import jax
import jax.numpy as jnp
from jax import lax
from jax.experimental import pallas as pl
from jax.experimental.pallas import tpu as pltpu
from jax.experimental.pallas import tpu_sc as plsc

_NC = 2
_NS = 16
_NW = _NC * _NS
_B = 16384
_D = 64
_BPW = _B // _NW
_L = 16
_CHUNK = 128
_NCHUNK = _BPW // _CHUNK

_MESH = plsc.VectorSubcoreMesh(core_axis_name="c", subcore_axis_name="s")
_PARAMS = pltpu.CompilerParams(
    needs_layout_passes=False, use_tc_tiling_on_sc=False
)


def _build_item_gather():
    @functools.partial(
        pl.kernel,
        out_type=jax.ShapeDtypeStruct((_B, _D), jnp.float32),
        mesh=_MESH,
        scratch_types=[
            pltpu.VMEM((_NCHUNK, _CHUNK), jnp.int32),
            pltpu.VMEM((_BPW, _D), jnp.float32),
            pltpu.SemaphoreType.DMA,
        ],
        compiler_params=_PARAMS,
    )
    def run(item_h, it_h, out_h, iidx, irows, sem):
        wid = lax.axis_index("s") * _NC + lax.axis_index("c")
        base = wid * _BPW
        for j in range(_NCHUNK):
            pltpu.sync_copy(item_h.at[pl.ds(base + j * _CHUNK, _CHUNK)], iidx.at[j])
        handles = [
            pltpu.async_copy(
                it_h.at[iidx.at[j]], irows.at[pl.ds(j * _CHUNK, _CHUNK)], sem
            )
            for j in range(_NCHUNK)
        ]
        for h in handles:
            h.wait()
        pltpu.sync_copy(irows, out_h.at[pl.ds(base, _BPW)])

    return run


def _build_main():
    @functools.partial(
        pl.kernel,
        out_type=jax.ShapeDtypeStruct((_B,), jnp.float32),
        mesh=_MESH,
        scratch_types=[
            pltpu.VMEM((_NCHUNK, _CHUNK), jnp.int32),
            pltpu.VMEM((_BPW, _D), jnp.float32),
            pltpu.VMEM((_BPW, _D), jnp.float32),
            pltpu.VMEM((_L, 17), jnp.float32),
            pltpu.VMEM((_BPW,), jnp.float32),
            pltpu.SemaphoreType.DMA,
            pltpu.SemaphoreType.DMA,
            pltpu.SemaphoreType.DMA,
            pltpu.SemaphoreType.DMA,
            pltpu.SemaphoreType.DMA,
        ],
        compiler_params=_PARAMS,
    )
    def run(user_h, irows_h, ut_h, out_h, uidx, urows, irows, smat, outv,
            isem, *sems):
        wid = lax.axis_index("s") * _NC + lax.axis_index("c")
        base = wid * _BPW

        ih = pltpu.async_copy(irows_h.at[pl.ds(base, _BPW)], irows, isem)
        for j in range(_NCHUNK):
            pltpu.sync_copy(user_h.at[pl.ds(base + j * _CHUNK, _CHUNK)], uidx.at[j])

        handles = [
            pltpu.async_copy(
                ut_h.at[uidx.at[j]], urows.at[pl.ds(j * _CHUNK, _CHUNK)], sems[j]
            )
            for j in range(_NCHUNK)
        ]
        ih.wait()

        lanes = lax.iota(jnp.int32, _L)

        def group(g):
            rbase = g * _L
            for r in range(_L):
                s = None
                for c in range(_D // _L):
                    u = urows[rbase + r, pl.ds(c * _L, _L)]
                    v = irows[rbase + r, pl.ds(c * _L, _L)]
                    s = u * v if s is None else s + u * v
                smat[r, pl.ds(0, _L)] = s
            acc = jnp.zeros((_L,), jnp.float32)
            for k in range(_L):
                col = plsc.load_gather(smat, [lanes, jnp.full((_L,), k, jnp.int32)])
                acc = acc + col
            outv[pl.ds(rbase, _L)] = acc

        for j in range(_NCHUNK):
            handles[j].wait()

            def chunk_body(gg, carry, j=j):
                group(j * (_CHUNK // _L) + gg)
                return carry

            lax.fori_loop(0, _CHUNK // _L, chunk_body, 0)

        pltpu.sync_copy(outv, out_h.at[pl.ds(base, _BPW)])

    return run


_ITEM_GATHER = _build_item_gather()
_MAIN = _build_main()


def kernel(user, item, user_table, item_table):
    item_rows = _ITEM_GATHER(item.astype(jnp.int32), item_table)
    return _MAIN(user.astype(jnp.int32), item_rows, user_table)


# speedup vs baseline: 1.5270x; 1.1582x over previous
import functools

import jax
import jax.numpy as jnp
from jax import lax
from jax.experimental import pallas as pl
from jax.experimental.pallas import tpu as pltpu
from jax.experimental.pallas import tpu_sc as plsc

_NC = 2
_NS = 16
_NW = _NC * _NS
_B = 16384
_D = 64
_BPW = _B // _NW
_L = 16
_NG = _BPW // _L
_TR = 8
_NT = 100000 // _TR


def _build():
    mesh = plsc.VectorSubcoreMesh(core_axis_name="c", subcore_axis_name="s")

    @functools.partial(
        pl.kernel,
        out_type=jax.ShapeDtypeStruct((_B,), jnp.float32),
        mesh=mesh,
        scratch_types=[
            pltpu.VMEM((_BPW,), jnp.int32),
            pltpu.VMEM((_BPW,), jnp.int32),
            pltpu.VMEM((2, _L * _TR, _D), jnp.float32),
            pltpu.VMEM((2, _L * _TR, _D), jnp.float32),
            pltpu.VMEM((_L, 17), jnp.float32),
            pltpu.VMEM((_BPW,), jnp.float32),
            pltpu.SemaphoreType.DMA,
            pltpu.SemaphoreType.DMA,
        ],
        compiler_params=pltpu.CompilerParams(needs_layout_passes=False),
    )
    def run(user_h, item_h, ut_h, it_h, out_h, uidx, iidx, ubuf, ibuf, smat,
            outv, sem0, sem1):
        sems = (sem0, sem1)
        wid = lax.axis_index("s") * _NC + lax.axis_index("c")
        base = wid * _BPW

        pltpu.sync_copy(user_h.at[pl.ds(base, _BPW)], uidx)
        pltpu.sync_copy(item_h.at[pl.ds(base, _BPW)], iidx)

        lanes = lax.iota(jnp.int32, _L)

        def issue(g, slot):
            uvec = uidx[pl.ds(g * _L, _L)]
            ivec = iidx[pl.ds(g * _L, _L)]
            ut = uvec >> 3
            it = ivec >> 3
            for r in range(_L):
                pltpu.async_copy(
                    ut_h.at[ut[r]],
                    ubuf.at[slot, pl.ds(r * _TR, _TR)],
                    sems[slot],
                )
                pltpu.async_copy(
                    it_h.at[it[r]],
                    ibuf.at[slot, pl.ds(r * _TR, _TR)],
                    sems[slot],
                )

        def drain(slot):
            pltpu.make_async_copy(
                ut_h.at[pl.ds(0, _L)], ubuf.at[slot], sems[slot]
            ).wait()
            pltpu.make_async_copy(
                it_h.at[pl.ds(0, _L)], ibuf.at[slot], sems[slot]
            ).wait()

        def compute(g, slot):
            uvec = uidx[pl.ds(g * _L, _L)]
            ivec = iidx[pl.ds(g * _L, _L)]
            us = uvec & (_TR - 1)
            is_ = ivec & (_TR - 1)
            for r in range(_L):
                ru = r * _TR + us[r]
                ri = r * _TR + is_[r]
                s = None
                for c in range(_D // _L):
                    u = ubuf[slot, ru, pl.ds(c * _L, _L)]
                    v = ibuf[slot, ri, pl.ds(c * _L, _L)]
                    s = u * v if s is None else s + u * v
                smat[r, pl.ds(0, _L)] = s
            acc = jnp.zeros((_L,), jnp.float32)
            for k in range(_L):
                col = plsc.load_gather(
                    smat, [lanes, jnp.full((_L,), k, jnp.int32)]
                )
                acc = acc + col
            outv[pl.ds(g * _L, _L)] = acc

        issue(jnp.int32(0), 0)
        issue(jnp.int32(1), 1)

        def body(k, carry):
            ge = 2 * k
            drain(0)
            compute(ge, 0)
            issue(ge + 2, 0)
            drain(1)
            compute(ge + 1, 1)
            issue(ge + 3, 1)
            return carry

        lax.fori_loop(0, _NG // 2 - 1, body, 0)

        drain(0)
        compute(jnp.int32(_NG - 2), 0)
        drain(1)
        compute(jnp.int32(_NG - 1), 1)

        pltpu.sync_copy(outv, out_h.at[pl.ds(base, _BPW)])

    return run


_KERNEL = _build()


def kernel(user, item, user_table, item_table):
    ut = user_table.reshape(_NT, _TR, _D)
    it = item_table.reshape(_NT, _TR, _D)
    return _KERNEL(
        user.astype(jnp.int32),
        item.astype(jnp.int32),
        ut,
        it,
    )
